# Initial kernel scaffold; baseline (speedup 1.0000x reference)
import functools

import jax
import jax.numpy as jnp
from jax import lax
from jax.experimental import pallas as pl
from jax.experimental.pallas import tpu as pltpu
from jax.experimental.pallas import tpu_sc as plsc

N_NODES = 10000
N_EDGES = 160000
N_ATOM_BASIS = 256
N_FILTERS = 256
N_GAUSSIANS = 64
CUTOFF = 5.0
LOG2 = 0.6931471805599453

HALF = N_FILTERS // 2

NT = 16
EPT = N_EDGES // NT
EB = 80
NB = EPT // EB
ROWS_PT = N_NODES // NT

BE = 2000
BN = 2000


def _ssp(x):
    return jax.nn.softplus(x) - LOG2


def _rf_body(r_ref, waf_ref, lo_ref, hi_ref):
    rf = jnp.dot(r_ref[...], waf_ref[...], preferred_element_type=jnp.float32)
    lo_ref[...] = rf[:, :HALF]
    hi_ref[...] = rf[:, HALF:]


def _rf_call(r, w_af):
    grid = (N_NODES // BN,)
    return pl.pallas_call(
        _rf_body,
        grid=grid,
        in_specs=[
            pl.BlockSpec((BN, N_ATOM_BASIS), lambda i: (i, 0)),
            pl.BlockSpec((N_ATOM_BASIS, N_FILTERS), lambda i: (0, 0)),
        ],
        out_specs=[
            pl.BlockSpec((BN, HALF), lambda i: (i, 0)),
            pl.BlockSpec((BN, HALF), lambda i: (i, 0)),
        ],
        out_shape=[
            jax.ShapeDtypeStruct((N_NODES, HALF), jnp.float32),
            jax.ShapeDtypeStruct((N_NODES, HALF), jnp.float32),
        ],
    )(r, w_af)


def _edge_mlp_body(e_ref, w1_ref, b1_ref, w2_ref, b2_ref, lo_ref, hi_ref):
    e = e_ref[...]
    step = CUTOFF / (N_GAUSSIANS - 1)
    offs = lax.broadcasted_iota(jnp.float32, (1, N_GAUSSIANS), 1) * step
    diff = e - offs
    coeff = -0.5 / (step * step)
    eg = jnp.exp(coeff * diff * diff)
    h = jnp.dot(eg, w1_ref[...], preferred_element_type=jnp.float32) + b1_ref[...]
    h = _ssp(h)
    w = jnp.dot(h, w2_ref[...], preferred_element_type=jnp.float32) + b2_ref[...]
    lo_ref[...] = w[:, :HALF]
    hi_ref[...] = w[:, HALF:]


def _edge_mlp_call(e, w_df1, b_df1, w_df2, b_df2):
    grid = (N_EDGES // BE,)
    return pl.pallas_call(
        _edge_mlp_body,
        grid=grid,
        in_specs=[
            pl.BlockSpec((BE, 1), lambda i: (i, 0)),
            pl.BlockSpec((N_GAUSSIANS, N_GAUSSIANS), lambda i: (0, 0)),
            pl.BlockSpec((1, N_GAUSSIANS), lambda i: (0, 0)),
            pl.BlockSpec((N_GAUSSIANS, N_FILTERS), lambda i: (0, 0)),
            pl.BlockSpec((1, N_FILTERS), lambda i: (0, 0)),
        ],
        out_specs=[
            pl.BlockSpec((BE, HALF), lambda i: (i, 0)),
            pl.BlockSpec((BE, HALF), lambda i: (i, 0)),
        ],
        out_shape=[
            jax.ShapeDtypeStruct((N_EDGES, HALF), jnp.float32),
            jax.ShapeDtypeStruct((N_EDGES, HALF), jnp.float32),
        ],
    )(e, w_df1, b_df1, w_df2, b_df2)


def _sc_body(rfl, rfh, wl, wh, es, ed, yl, yh,
             src_v, dst_v, rfs_v, rfd_v, w_v, ybuf_v, acc_sh, sem1, sem2):
    c = lax.axis_index("c")
    s = lax.axis_index("s")

    def zero_row(i, carry):
        for j in range(HALF // 16):
            ybuf_v[i, pl.ds(j * 16, 16)] = jnp.zeros((16,), jnp.float32)
        return carry

    lax.fori_loop(0, ROWS_PT, zero_row, 0)
    pltpu.sync_copy(ybuf_v, acc_sh.at[pl.ds(s * ROWS_PT, ROWS_PT)])
    plsc.subcore_barrier()

    def work(rf_hbm, w_hbm, y_hbm):
        def blk_body(b, carry):
            base = s * EPT + b * EB
            pltpu.sync_copy(es.at[pl.ds(base, EB)], src_v)
            pltpu.sync_copy(ed.at[pl.ds(base, EB)], dst_v)
            cp1 = pltpu.async_copy(rf_hbm.at[src_v], rfs_v, sem1)
            cp2 = pltpu.async_copy(rf_hbm.at[dst_v], rfd_v, sem2)
            pltpu.sync_copy(w_hbm.at[pl.ds(base, EB)], w_v)
            cp1.wait()
            cp2.wait()

            def mul_body(i, carry2):
                for j in range(HALF // 16):
                    sl = pl.ds(j * 16, 16)
                    wv = w_v[i, sl]
                    rfs_v[i, sl] = rfs_v[i, sl] * wv
                    rfd_v[i, sl] = rfd_v[i, sl] * wv
                return carry2

            lax.fori_loop(0, EB, mul_body, 0)
            pltpu.sync_copy(rfs_v, acc_sh.at[dst_v], add=True)
            pltpu.sync_copy(rfd_v, acc_sh.at[src_v], add=True)
            return carry

        lax.fori_loop(0, NB, blk_body, 0)
        plsc.subcore_barrier()
        pltpu.sync_copy(acc_sh.at[pl.ds(s * ROWS_PT, ROWS_PT)], ybuf_v)
        pltpu.sync_copy(ybuf_v, y_hbm.at[pl.ds(s * ROWS_PT, ROWS_PT)])

    @pl.when(c == 0)
    def _():
        work(rfl, wl, yl)

    @pl.when(c == 1)
    def _():
        work(rfh, wh, yh)


def _sc_call(rf_lo, rf_hi, w_lo, w_hi, src, dst):
    mesh = plsc.VectorSubcoreMesh(core_axis_name="c", subcore_axis_name="s")
    f = functools.partial(
        pl.kernel,
        out_type=[
            jax.ShapeDtypeStruct((N_NODES, HALF), jnp.float32),
            jax.ShapeDtypeStruct((N_NODES, HALF), jnp.float32),
        ],
        mesh=mesh,
        scratch_types=[
            pltpu.VMEM((EB,), jnp.int32),
            pltpu.VMEM((EB,), jnp.int32),
            pltpu.VMEM((EB, HALF), jnp.float32),
            pltpu.VMEM((EB, HALF), jnp.float32),
            pltpu.VMEM((EB, HALF), jnp.float32),
            pltpu.VMEM((ROWS_PT, HALF), jnp.float32),
            pltpu.VMEM_SHARED((N_NODES, HALF), jnp.float32),
            pltpu.SemaphoreType.DMA,
            pltpu.SemaphoreType.DMA,
        ],
    )(_sc_body)
    return f(rf_lo, rf_hi, w_lo, w_hi, src, dst)


def _out_mlp_body(ylo_ref, yhi_ref, w1_ref, b1_ref, w2_ref, b2_ref, o_ref):
    y = jnp.concatenate([ylo_ref[...], yhi_ref[...]], axis=1)
    h = jnp.dot(y, w1_ref[...], preferred_element_type=jnp.float32) + b1_ref[...]
    h = _ssp(h)
    o_ref[...] = jnp.dot(h, w2_ref[...], preferred_element_type=jnp.float32) + b2_ref[...]


def _out_mlp_call(y_lo, y_hi, w_d1, b_d1, w_d2, b_d2):
    grid = (N_NODES // BN,)
    return pl.pallas_call(
        _out_mlp_body,
        grid=grid,
        in_specs=[
            pl.BlockSpec((BN, HALF), lambda i: (i, 0)),
            pl.BlockSpec((BN, HALF), lambda i: (i, 0)),
            pl.BlockSpec((N_FILTERS, N_ATOM_BASIS), lambda i: (0, 0)),
            pl.BlockSpec((1, N_ATOM_BASIS), lambda i: (0, 0)),
            pl.BlockSpec((N_ATOM_BASIS, N_ATOM_BASIS), lambda i: (0, 0)),
            pl.BlockSpec((1, N_ATOM_BASIS), lambda i: (0, 0)),
        ],
        out_specs=pl.BlockSpec((BN, N_ATOM_BASIS), lambda i: (i, 0)),
        out_shape=jax.ShapeDtypeStruct((N_NODES, N_ATOM_BASIS), jnp.float32),
    )(y_lo, y_hi, w_d1, b_d1, w_d2, b_d2)


def kernel(r, e, a, W_df1, b_df1, W_df2, b_df2, W_af, W_d1, b_d1, W_d2, b_d2):
    src = a[:, 0]
    dst = a[:, 1]
    rf_lo, rf_hi = _rf_call(r, W_af)
    w_lo, w_hi = _edge_mlp_call(e, W_df1, b_df1.reshape(1, -1),
                                W_df2, b_df2.reshape(1, -1))
    y_lo, y_hi = _sc_call(rf_lo, rf_hi, w_lo, w_hi, src, dst)
    return _out_mlp_call(y_lo, y_hi, W_d1, b_d1.reshape(1, -1),
                         W_d2, b_d2.reshape(1, -1))


# speedup vs baseline: 2.9938x; 2.9938x over previous
import functools

import jax
import jax.numpy as jnp
from jax import lax
from jax.experimental import pallas as pl
from jax.experimental.pallas import tpu as pltpu
from jax.experimental.pallas import tpu_sc as plsc

N_NODES = 10000
N_EDGES = 160000
N_ATOM_BASIS = 256
N_FILTERS = 256
N_GAUSSIANS = 64
CUTOFF = 5.0
LOG2 = 0.6931471805599453

HALF = N_FILTERS // 2

NT = 16
EPT = N_EDGES // NT
EB = 80
NB = EPT // EB
ROWS_PT = 640
N_PAD = ROWS_PT * NT

BE = 2000
BN = 2000


def _ssp(x):
    return jax.nn.softplus(x) - LOG2


def _rf_body(r_ref, waf_ref, lo_ref, hi_ref):
    rf = jnp.dot(r_ref[...], waf_ref[...], preferred_element_type=jnp.float32)
    lo_ref[...] = rf[:, :HALF]
    hi_ref[...] = rf[:, HALF:]


def _rf_call(r, w_af):
    grid = (N_NODES // BN,)
    return pl.pallas_call(
        _rf_body,
        grid=grid,
        in_specs=[
            pl.BlockSpec((BN, N_ATOM_BASIS), lambda i: (i, 0)),
            pl.BlockSpec((N_ATOM_BASIS, N_FILTERS), lambda i: (0, 0)),
        ],
        out_specs=[
            pl.BlockSpec((BN, HALF), lambda i: (i, 0)),
            pl.BlockSpec((BN, HALF), lambda i: (i, 0)),
        ],
        out_shape=[
            jax.ShapeDtypeStruct((N_NODES, HALF), jnp.float32),
            jax.ShapeDtypeStruct((N_NODES, HALF), jnp.float32),
        ],
    )(r, w_af)


def _edge_mlp_body(e_ref, w1_ref, b1_ref, w2_ref, b2_ref, lo_ref, hi_ref):
    e = e_ref[...]
    step = CUTOFF / (N_GAUSSIANS - 1)
    offs = lax.broadcasted_iota(jnp.int32, (1, N_GAUSSIANS), 1).astype(jnp.float32) * step
    diff = e - offs
    coeff = -0.5 / (step * step)
    eg = jnp.exp(coeff * diff * diff)
    h = jnp.dot(eg, w1_ref[...], preferred_element_type=jnp.float32) + b1_ref[...]
    h = _ssp(h)
    w = jnp.dot(h, w2_ref[...], preferred_element_type=jnp.float32) + b2_ref[...]
    lo_ref[...] = w[:, :HALF]
    hi_ref[...] = w[:, HALF:]


def _edge_mlp_call(e, w_df1, b_df1, w_df2, b_df2):
    grid = (N_EDGES // BE,)
    return pl.pallas_call(
        _edge_mlp_body,
        grid=grid,
        in_specs=[
            pl.BlockSpec((BE, 1), lambda i: (i, 0)),
            pl.BlockSpec((N_GAUSSIANS, N_GAUSSIANS), lambda i: (0, 0)),
            pl.BlockSpec((1, N_GAUSSIANS), lambda i: (0, 0)),
            pl.BlockSpec((N_GAUSSIANS, N_FILTERS), lambda i: (0, 0)),
            pl.BlockSpec((1, N_FILTERS), lambda i: (0, 0)),
        ],
        out_specs=[
            pl.BlockSpec((BE, HALF), lambda i: (i, 0)),
            pl.BlockSpec((BE, HALF), lambda i: (i, 0)),
        ],
        out_shape=[
            jax.ShapeDtypeStruct((N_EDGES, HALF), jnp.float32),
            jax.ShapeDtypeStruct((N_EDGES, HALF), jnp.float32),
        ],
    )(e, w_df1, b_df1, w_df2, b_df2)


def _sc_body(rfl, rfh, wl, wh, es, ed, yl, yh,
             src_v, dst_v, rfs_v, rfd_v, w_v, acc_sh, sem1, sem2):
    c = lax.axis_index("c")
    s = lax.axis_index("s")

    def zero_row(i, carry):
        for j in range(HALF // 16):
            rfs_v[i, pl.ds(j * 16, 16)] = jnp.zeros((16,), jnp.float32)
        return carry

    lax.fori_loop(0, EB, zero_row, 0)

    def zero_chunk(k, carry):
        pltpu.sync_copy(rfs_v, acc_sh.at[pl.ds(s * ROWS_PT + k * EB, EB)])
        return carry

    lax.fori_loop(0, ROWS_PT // EB, zero_chunk, 0)
    plsc.subcore_barrier()

    def work(rf_hbm, w_hbm, y_hbm):
        def blk_body(b, carry):
            base = s * EPT + b * EB
            pltpu.sync_copy(es.at[pl.ds(base, EB)], src_v)
            pltpu.sync_copy(ed.at[pl.ds(base, EB)], dst_v)
            cp1 = pltpu.async_copy(rf_hbm.at[src_v], rfs_v, sem1)
            cp2 = pltpu.async_copy(rf_hbm.at[dst_v], rfd_v, sem2)
            pltpu.sync_copy(w_hbm.at[pl.ds(base, EB)], w_v)
            cp1.wait()
            cp2.wait()

            def mul_body(i, carry2):
                for j in range(HALF // 16):
                    sl = pl.ds(j * 16, 16)
                    wv = w_v[i, sl]
                    rfs_v[i, sl] = rfs_v[i, sl] * wv
                    rfd_v[i, sl] = rfd_v[i, sl] * wv
                return carry2

            lax.fori_loop(0, EB, mul_body, 0)
            pltpu.sync_copy(rfs_v, acc_sh.at[dst_v], add=True)
            pltpu.sync_copy(rfd_v, acc_sh.at[src_v], add=True)
            return carry

        lax.fori_loop(0, NB, blk_body, 0)
        plsc.subcore_barrier()

        def out_chunk(k, carry):
            base = s * ROWS_PT + k * EB
            pltpu.sync_copy(acc_sh.at[pl.ds(base, EB)], rfs_v)
            pltpu.sync_copy(rfs_v, y_hbm.at[pl.ds(base, EB)])
            return carry

        lax.fori_loop(0, ROWS_PT // EB, out_chunk, 0)

    @pl.when(c == 0)
    def _():
        work(rfl, wl, yl)

    @pl.when(c == 1)
    def _():
        work(rfh, wh, yh)


def _sc_call(rf_lo, rf_hi, w_lo, w_hi, src, dst):
    mesh = plsc.VectorSubcoreMesh(core_axis_name="c", subcore_axis_name="s",
                                  num_cores=2, num_subcores=NT)
    f = functools.partial(
        pl.kernel,
        out_type=[
            jax.ShapeDtypeStruct((N_PAD, HALF), jnp.float32),
            jax.ShapeDtypeStruct((N_PAD, HALF), jnp.float32),
        ],
        mesh=mesh,
        scratch_types=[
            pltpu.VMEM((EB,), jnp.int32),
            pltpu.VMEM((EB,), jnp.int32),
            pltpu.VMEM((EB, HALF), jnp.float32),
            pltpu.VMEM((EB, HALF), jnp.float32),
            pltpu.VMEM((EB, HALF), jnp.float32),
            pltpu.VMEM_SHARED((N_PAD, HALF), jnp.float32),
            pltpu.SemaphoreType.DMA,
            pltpu.SemaphoreType.DMA,
        ],
    )(_sc_body)
    return f(rf_lo, rf_hi, w_lo, w_hi, src, dst)


def _out_mlp_body(ylo_ref, yhi_ref, w1_ref, b1_ref, w2_ref, b2_ref, o_ref):
    y = jnp.concatenate([ylo_ref[...], yhi_ref[...]], axis=1)
    h = jnp.dot(y, w1_ref[...], preferred_element_type=jnp.float32) + b1_ref[...]
    h = _ssp(h)
    o_ref[...] = jnp.dot(h, w2_ref[...], preferred_element_type=jnp.float32) + b2_ref[...]


def _out_mlp_call(y_lo, y_hi, w_d1, b_d1, w_d2, b_d2):
    grid = (N_NODES // BN,)
    return pl.pallas_call(
        _out_mlp_body,
        grid=grid,
        in_specs=[
            pl.BlockSpec((BN, HALF), lambda i: (i, 0)),
            pl.BlockSpec((BN, HALF), lambda i: (i, 0)),
            pl.BlockSpec((N_FILTERS, N_ATOM_BASIS), lambda i: (0, 0)),
            pl.BlockSpec((1, N_ATOM_BASIS), lambda i: (0, 0)),
            pl.BlockSpec((N_ATOM_BASIS, N_ATOM_BASIS), lambda i: (0, 0)),
            pl.BlockSpec((1, N_ATOM_BASIS), lambda i: (0, 0)),
        ],
        out_specs=pl.BlockSpec((BN, N_ATOM_BASIS), lambda i: (i, 0)),
        out_shape=jax.ShapeDtypeStruct((N_NODES, N_ATOM_BASIS), jnp.float32),
    )(y_lo, y_hi, w_d1, b_d1, w_d2, b_d2)


def kernel(r, e, a, W_df1, b_df1, W_df2, b_df2, W_af, W_d1, b_d1, W_d2, b_d2):
    src = a[:, 0]
    dst = a[:, 1]
    rf_lo, rf_hi = _rf_call(r, W_af)
    w_lo, w_hi = _edge_mlp_call(e, W_df1, b_df1.reshape(1, -1),
                                W_df2, b_df2.reshape(1, -1))
    y_lo, y_hi = _sc_call(rf_lo, rf_hi, w_lo, w_hi, src, dst)
    y_lo = y_lo[:N_NODES]
    y_hi = y_hi[:N_NODES]
    return _out_mlp_call(y_lo, y_hi, W_d1, b_d1.reshape(1, -1),
                         W_d2, b_d2.reshape(1, -1))
